# baseline (speedup 1.0000x reference)
import functools

import jax
import jax.numpy as jnp
from jax import lax
from jax.experimental import pallas as pl
from jax.experimental.pallas import tpu as pltpu
from jax.experimental.pallas import tpu_sc as plsc

N_NEURONS = 2048
D_MODEL = 256
RANK = 16

@functools.lru_cache(maxsize=None)
def _make_gather(n_idx: int):
    info = plsc.get_sparse_core_info()
    nw = info.num_cores * info.num_subcores
    bpw = n_idx // nw
    mesh = plsc.VectorSubcoreMesh(core_axis_name="c", subcore_axis_name="s")

    chunks = []
    off = 0
    while off < bpw:
        sz = 16 if len(chunks) % 2 == 0 else 8
        sz = min(sz, bpw - off)
        chunks.append((off, sz))
        off += sz

    @functools.partial(
        pl.kernel,
        mesh=mesh,
        out_type=[jax.ShapeDtypeStruct((n_idx, RANK, D_MODEL), jnp.float32)] * 4,
        scratch_types=[
            pltpu.VMEM((bpw,), jnp.int32),
            pltpu.VMEM((16, RANK, D_MODEL), jnp.float32),
            pltpu.VMEM((8, RANK, D_MODEL), jnp.float32),
            pltpu.SemaphoreType.DMA,
            pltpu.SemaphoreType.DMA,
        ],
    )
    def run(idx_hbm, q_hbm, k_hbm, v_hbm, o_hbm,
            oq_hbm, ok_hbm, ov_hbm, oo_hbm,
            idx_v, buf_big, buf_small, sem0, sem1):
        wid = lax.axis_index("s") * info.num_cores + lax.axis_index("c")
        base = wid * bpw
        pltpu.sync_copy(idx_hbm.at[pl.ds(base, bpw)], idx_v)

        def dst_of(c):
            coff, sz = chunks[c]
            buf = buf_big if c % 2 == 0 else buf_small
            if sz < (16 if c % 2 == 0 else 8):
                buf = buf.at[pl.ds(0, sz)]
            return buf

        def sem_of(c):
            return sem0 if c % 2 == 0 else sem1

        def gather(tbl, c):
            coff, sz = chunks[c]
            pltpu.async_copy(tbl.at[idx_v.at[pl.ds(coff, sz)]], dst_of(c),
                             sem_of(c))

        def drain(tbl, c):
            coff, sz = chunks[c]
            pltpu.make_async_copy(tbl.at[idx_v.at[pl.ds(coff, sz)]], dst_of(c),
                                  sem_of(c)).wait()

        def put(out, c):
            coff, sz = chunks[c]
            pltpu.sync_copy(dst_of(c), out.at[pl.ds(base + coff, sz)])

        nchk = len(chunks)
        for tbl, out in ((q_hbm, oq_hbm), (k_hbm, ok_hbm),
                         (v_hbm, ov_hbm), (o_hbm, oo_hbm)):
            gather(tbl, 0)
            for c in range(nchk):
                if c + 1 < nchk:
                    gather(tbl, c + 1)
                drain(tbl, c)
                put(out, c)

    return run


def kernel(indices, W_Q, W_K, W_V, W_O):
    b, s, k = indices.shape
    n_idx = b * s * k
    idx = indices.reshape(n_idx).astype(jnp.int32)
    banks = [jnp.swapaxes(w, 1, 2) for w in (W_Q, W_K, W_V)] + [W_O]
    oq, ok, ov, oo = _make_gather(n_idx)(idx, *banks)
    return (
        jnp.swapaxes(oq, 1, 2).reshape(b, s, k, D_MODEL, RANK),
        jnp.swapaxes(ok, 1, 2).reshape(b, s, k, D_MODEL, RANK),
        jnp.swapaxes(ov, 1, 2).reshape(b, s, k, D_MODEL, RANK),
        oo.reshape(b, s, k, RANK, D_MODEL),
    )


# speedup vs baseline: 1.0273x; 1.0273x over previous
import functools

import jax
import jax.numpy as jnp
from jax import lax
from jax.experimental import pallas as pl
from jax.experimental.pallas import tpu as pltpu
from jax.experimental.pallas import tpu_sc as plsc

N_NEURONS = 2048
D_MODEL = 256
RANK = 16

CHUNK = 8


@functools.lru_cache(maxsize=None)
def _make_gather(n_idx: int):
    info = plsc.get_sparse_core_info()
    nw = info.num_cores * info.num_subcores
    bpw = n_idx // nw
    nch = bpw // CHUNK
    pairs = nch // 2
    mesh = plsc.VectorSubcoreMesh(core_axis_name="c", subcore_axis_name="s")

    @functools.partial(
        pl.kernel,
        mesh=mesh,
        out_type=[jax.ShapeDtypeStruct((n_idx, RANK, D_MODEL), jnp.float32)] * 4,
        scratch_types=[
            pltpu.VMEM((bpw,), jnp.int32),
            pltpu.VMEM((CHUNK, RANK, D_MODEL), jnp.float32),
            pltpu.VMEM((CHUNK, RANK, D_MODEL), jnp.float32),
            pltpu.SemaphoreType.DMA,
            pltpu.SemaphoreType.DMA,
        ],
    )
    def run(idx_hbm, q_hbm, k_hbm, v_hbm, o_hbm,
            oq_hbm, ok_hbm, ov_hbm, oo_hbm,
            idx_v, buf0, buf1, sem0, sem1):
        wid = lax.axis_index("s") * info.num_cores + lax.axis_index("c")
        base = wid * bpw
        pltpu.sync_copy(idx_hbm.at[pl.ds(base, bpw)], idx_v)

        def gather(tbl, dst, sem, c):
            pltpu.async_copy(tbl.at[idx_v.at[pl.ds(c * CHUNK, CHUNK)]], dst, sem)

        def drain(tbl, dst, sem):
            pltpu.make_async_copy(tbl.at[idx_v.at[pl.ds(0, CHUNK)]], dst, sem).wait()

        def put(out, buf, c):
            pltpu.sync_copy(buf, out.at[pl.ds(base + c * CHUNK, CHUNK)])

        for tbl, out in ((q_hbm, oq_hbm), (k_hbm, ok_hbm),
                         (v_hbm, ov_hbm), (o_hbm, oo_hbm)):
            gather(tbl, buf0, sem0, 0)

            def body(i, carry, tbl=tbl, out=out):
                c = 2 * i
                gather(tbl, buf1, sem1, c + 1)
                drain(tbl, buf0, sem0)
                put(out, buf0, c)

                @pl.when(i < pairs - 1)
                def _():
                    gather(tbl, buf0, sem0, c + 2)

                drain(tbl, buf1, sem1)
                put(out, buf1, c + 1)
                return carry

            lax.fori_loop(0, pairs, body, 0)

    return run


def kernel(indices, W_Q, W_K, W_V, W_O):
    b, s, k = indices.shape
    n_idx = b * s * k
    idx = indices.reshape(n_idx).astype(jnp.int32)
    banks = [jnp.swapaxes(w, 1, 2) for w in (W_Q, W_K, W_V)] + [W_O]
    oq, ok, ov, oo = _make_gather(n_idx)(idx, *banks)
    return (
        jnp.swapaxes(oq, 1, 2).reshape(b, s, k, D_MODEL, RANK),
        jnp.swapaxes(ok, 1, 2).reshape(b, s, k, D_MODEL, RANK),
        jnp.swapaxes(ov, 1, 2).reshape(b, s, k, D_MODEL, RANK),
        oo.reshape(b, s, k, RANK, D_MODEL),
    )
